# baseline (speedup 1.0000x reference)
import jax
import jax.numpy as jnp
from jax.experimental import pallas as pl
from jax.experimental.pallas import tpu as pltpu

_C = 21
_NOBJ = 8
_R = 8


def _match_body(boxes_ref, vals_ref, priors_ref, loc_ref,
                pos_ref, tc_ref, pm_ref, pext_ref):
    i = pl.program_id(0)
    P = priors_ref.shape[1]

    @pl.when(i == 0)
    def _prep():
        pr = priors_ref[...]
        cx, cy, w, h = pr[0:1], pr[1:2], pr[2:3], pr[3:4]
        x0 = cx - w / 2.
        y0 = cy - h / 2.
        x1 = cx + w / 2.
        y1 = cy + h / 2.
        pext_ref[...] = jnp.concatenate(
            [cx, cy, w / 10., h / 10., x0, y0, x1, y1,
             (x1 - x0) * (y1 - y0), w], axis=0)

    pe = pext_ref[...]
    pcx, pcy, pw10, ph10 = pe[0:1], pe[1:2], pe[2:3], pe[3:4]
    px0, py0, px1, py1 = pe[4:5], pe[5:6], pe[6:7], pe[7:8]
    area_p = pe[8:9]
    pw, ph = pe[9:10], pe[3:4] * 10.

    for r in range(_R):
        b = boxes_ref[r]
        bx0, by0, bx1, by1 = b[:, 0:1], b[:, 1:2], b[:, 2:3], b[:, 3:4]
        area_b = (bx1 - bx0) * (by1 - by0)

        lt_x = jnp.maximum(bx0, px0)
        lt_y = jnp.maximum(by0, py0)
        rb_x = jnp.minimum(bx1, px1)
        rb_y = jnp.minimum(by1, py1)
        iw = jnp.maximum(rb_x - lt_x, 0.)
        ih = jnp.maximum(rb_y - lt_y, 0.)
        inter = iw * ih
        iou = inter / (area_b + area_p - inter)

        lane = jax.lax.broadcasted_iota(jnp.int32, (_NOBJ, P), 1)
        row_max = jnp.max(iou, axis=1, keepdims=True)
        best_p = jnp.min(jnp.where(iou == row_max, lane, P), axis=1,
                         keepdims=True)
        iou = jnp.where(lane == best_p, 1.0, iou)

        col_max = jnp.max(iou, axis=0, keepdims=True)
        pos = col_max >= 0.5
        oid = jax.lax.broadcasted_iota(jnp.int32, (_NOBJ, P), 0)
        sel = jnp.min(jnp.where(iou == col_max, oid, _NOBJ), axis=0,
                      keepdims=True)
        onehot = (oid == sel).astype(jnp.float32)

        g = jax.lax.dot_general(vals_ref[r], onehot,
                                (((1,), (0,)), ((), ())),
                                precision=jax.lax.Precision.HIGHEST,
                                preferred_element_type=jnp.float32)
        gx0, gy0, gx1, gy1 = g[0:1], g[1:2], g[2:3], g[3:4]
        tc = (g[4:5] + 0.5).astype(jnp.int32)
        tc = jnp.where(pos, tc, _C - 1)

        gcx = (gx0 + gx1) / 2.
        gcy = (gy0 + gy1) / 2.
        gw = gx1 - gx0
        gh = gy1 - gy0
        t0 = (gcx - pcx) / pw10
        t1 = (gcy - pcy) / ph10
        t2 = jnp.log(gw / pw) * 5.
        t3 = jnp.log(gh / ph) * 5.
        tgt = jnp.concatenate([t0, t1, t2, t3], axis=0)

        d = loc_ref[r] - tgt
        ad = jnp.abs(d)
        sl1 = jnp.where(ad < 1.0, 0.5 * d * d, ad - 0.5)
        posf = pos.astype(jnp.float32)
        loc_sum = jnp.sum(sl1 * posf)
        n_pos = jnp.sum(posf)

        pos_ref[r] = posf
        tc_ref[r] = tc
        pm_ref[r] = jnp.concatenate(
            [jnp.full((1, 1), loc_sum), jnp.full((1, 1), n_pos),
             jnp.zeros((1, 2))], axis=1)


def _ce_body(cls_ref, tc_ref, pos_ref, pm_ref, out_ref, ceneg_ref, row_ref):
    i = pl.program_id(0)
    nsteps = pl.num_programs(0)
    P = cls_ref.shape[2]
    B = ceneg_ref.shape[0]

    for r in range(_R):
        cls = cls_ref[r]
        tc = tc_ref[r]
        posf = pos_ref[r]

        m = jnp.max(cls, axis=0, keepdims=True)
        lse = jnp.log(jnp.sum(jnp.exp(cls - m), axis=0, keepdims=True))
        cid = jax.lax.broadcasted_iota(jnp.int32, (_C, P), 0)
        logit_tc = jnp.sum(jnp.where(cid == tc, cls, 0.), axis=0,
                           keepdims=True)
        ce = m + lse - logit_tc
        row_ref[_R * i + r, 0] = jnp.sum(ce * posf)
        ceneg_ref[pl.ds(_R * i + r, 1), :] = jnp.maximum(
            jnp.where(posf > 0.5, 0., ce), 0.)

    @pl.when(i == nsteps - 1)
    def _mine():
        v = ceneg_ref[...]
        bits = jax.lax.bitcast_convert_type(v, jnp.int32)
        rid = jax.lax.broadcasted_iota(jnp.int32, (B, 1), 0)
        conf_pos = jnp.zeros((B, 1), jnp.float32)
        for r in range(B):
            conf_pos = jnp.where(rid == r, row_ref[r, 0], conf_pos)
        pm = pm_ref[...]
        loc_sum = pm[:, 0, 0:1]
        n_pos = pm[:, 0, 1:2]
        k = jnp.minimum(3 * n_pos.astype(jnp.int32), P)

        def count_ge(t):
            return jnp.sum((bits >= t).astype(jnp.int32), axis=1,
                           keepdims=True)

        def step(_, carry):
            lo, hi = carry
            q = jnp.maximum((hi - lo) // 4, 1)
            m1 = lo + q
            m2 = lo + 2 * q
            m3 = hi - q
            c1 = count_ge(m1) >= k
            c2 = count_ge(m2) >= k
            c3 = count_ge(m3) >= k
            nlo = jnp.where(c3, m3, jnp.where(c2, m2, jnp.where(c1, m1, lo)))
            nhi = jnp.where(c3, hi, jnp.where(c2, m3, jnp.where(c1, m2, m1)))
            return (nlo, nhi)

        lo0 = jnp.zeros((B, 1), jnp.int32)
        hi0 = jnp.full((B, 1), 0x7f800000, jnp.int32)
        lo, hi = jax.lax.fori_loop(0, 18, step, (lo0, hi0))
        thr = jax.lax.bitcast_convert_type(lo, jnp.float32)
        gt = bits > lo
        cnt_gt = jnp.sum(gt.astype(jnp.int32), axis=1, keepdims=True)
        sum_gt = jnp.sum(jnp.where(gt, v, 0.), axis=1, keepdims=True)
        hard = sum_gt + (k - cnt_gt).astype(jnp.float32) * thr

        n_pos_sum = jnp.sum(n_pos)
        conf_loss = (jnp.sum(hard) + jnp.sum(conf_pos)) / n_pos_sum
        loc_loss = jnp.sum(loc_sum) / n_pos_sum
        total = conf_loss + loc_loss
        out_ref[...] = jnp.concatenate(
            [jnp.full((1, 1), total), jnp.full((1, 1), loc_loss),
             jnp.full((1, 1), conf_loss), jnp.zeros((1, 1))], axis=1)


@jax.jit
def kernel(pred_loc, pred_cls, b_boxes, b_labels, priors_cxcy):
    B, P, C = pred_cls.shape
    loc_t = jnp.transpose(pred_loc, (0, 2, 1))
    cls_t = jnp.transpose(pred_cls, (0, 2, 1))
    priors_t = jnp.transpose(priors_cxcy, (1, 0))
    vals = jnp.concatenate(
        [jnp.transpose(b_boxes, (0, 2, 1)),
         b_labels.astype(jnp.float32)[:, None, :]], axis=1)

    pos, tc, pm = pl.pallas_call(
        _match_body,
        grid=(B // _R,),
        in_specs=[
            pl.BlockSpec((_R, _NOBJ, 4), lambda i: (i, 0, 0)),
            pl.BlockSpec((_R, 5, _NOBJ), lambda i: (i, 0, 0)),
            pl.BlockSpec((4, P), lambda i: (0, 0)),
            pl.BlockSpec((_R, 4, P), lambda i: (i, 0, 0)),
        ],
        out_specs=[
            pl.BlockSpec((_R, 1, P), lambda i: (i, 0, 0)),
            pl.BlockSpec((_R, 1, P), lambda i: (i, 0, 0)),
            pl.BlockSpec((_R, 1, 4), lambda i: (i, 0, 0)),
        ],
        out_shape=[
            jax.ShapeDtypeStruct((B, 1, P), jnp.float32),
            jax.ShapeDtypeStruct((B, 1, P), jnp.int32),
            jax.ShapeDtypeStruct((B, 1, 4), jnp.float32),
        ],
        scratch_shapes=[pltpu.VMEM((10, P), jnp.float32)],
    )(b_boxes, vals, priors_t, loc_t)

    out = pl.pallas_call(
        _ce_body,
        grid=(B // _R,),
        in_specs=[
            pl.BlockSpec((_R, C, P), lambda i: (i, 0, 0)),
            pl.BlockSpec((_R, 1, P), lambda i: (i, 0, 0)),
            pl.BlockSpec((_R, 1, P), lambda i: (i, 0, 0)),
            pl.BlockSpec((B, 1, 4), lambda i: (0, 0, 0)),
        ],
        out_specs=pl.BlockSpec((1, 4), lambda i: (0, 0)),
        out_shape=jax.ShapeDtypeStruct((1, 4), jnp.float32),
        scratch_shapes=[
            pltpu.VMEM((B, P), jnp.float32),
            pltpu.SMEM((B, 4), jnp.float32),
        ],
    )(cls_t, tc, pos, pm)

    return (out[0, 0], out[0, 1], out[0, 2])


# speedup vs baseline: 1.0047x; 1.0047x over previous
import jax
import jax.numpy as jnp
from jax.experimental import pallas as pl
from jax.experimental.pallas import tpu as pltpu

_C = 21
_NOBJ = 8
_R = 4


def _match_body(boxes_ref, vals_ref, priors_ref, loc_ref,
                pos_ref, tc_ref, pm_ref, pext_ref):
    i = pl.program_id(0)
    P = priors_ref.shape[1]

    @pl.when(i == 0)
    def _prep():
        pr = priors_ref[...]
        cx, cy, w, h = pr[0:1], pr[1:2], pr[2:3], pr[3:4]
        x0 = cx - w / 2.
        y0 = cy - h / 2.
        x1 = cx + w / 2.
        y1 = cy + h / 2.
        pext_ref[...] = jnp.concatenate(
            [cx, cy, w / 10., h / 10., x0, y0, x1, y1,
             (x1 - x0) * (y1 - y0), w], axis=0)

    pe = pext_ref[...]
    pcx, pcy, pw10, ph10 = pe[0:1], pe[1:2], pe[2:3], pe[3:4]
    px0, py0, px1, py1 = pe[4:5], pe[5:6], pe[6:7], pe[7:8]
    area_p = pe[8:9]
    pw, ph = pe[9:10], pe[3:4] * 10.

    for r in range(_R):
        b = boxes_ref[r]
        bx0, by0, bx1, by1 = b[:, 0:1], b[:, 1:2], b[:, 2:3], b[:, 3:4]
        area_b = (bx1 - bx0) * (by1 - by0)

        lt_x = jnp.maximum(bx0, px0)
        lt_y = jnp.maximum(by0, py0)
        rb_x = jnp.minimum(bx1, px1)
        rb_y = jnp.minimum(by1, py1)
        iw = jnp.maximum(rb_x - lt_x, 0.)
        ih = jnp.maximum(rb_y - lt_y, 0.)
        inter = iw * ih
        iou = inter / (area_b + area_p - inter)

        lane = jax.lax.broadcasted_iota(jnp.int32, (_NOBJ, P), 1)
        row_max = jnp.max(iou, axis=1, keepdims=True)
        best_p = jnp.min(jnp.where(iou == row_max, lane, P), axis=1,
                         keepdims=True)
        iou = jnp.where(lane == best_p, 1.0, iou)

        col_max = jnp.max(iou, axis=0, keepdims=True)
        pos = col_max >= 0.5
        oid = jax.lax.broadcasted_iota(jnp.int32, (_NOBJ, P), 0)
        sel = jnp.min(jnp.where(iou == col_max, oid, _NOBJ), axis=0,
                      keepdims=True)
        onehot = (oid == sel).astype(jnp.float32)

        g = jax.lax.dot_general(vals_ref[r], onehot,
                                (((1,), (0,)), ((), ())),
                                precision=jax.lax.Precision.HIGHEST,
                                preferred_element_type=jnp.float32)
        gx0, gy0, gx1, gy1 = g[0:1], g[1:2], g[2:3], g[3:4]
        tc = (g[4:5] + 0.5).astype(jnp.int32)
        tc = jnp.where(pos, tc, _C - 1)

        gcx = (gx0 + gx1) / 2.
        gcy = (gy0 + gy1) / 2.
        gw = gx1 - gx0
        gh = gy1 - gy0
        t0 = (gcx - pcx) / pw10
        t1 = (gcy - pcy) / ph10
        t2 = jnp.log(gw / pw) * 5.
        t3 = jnp.log(gh / ph) * 5.
        tgt = jnp.concatenate([t0, t1, t2, t3], axis=0)

        d = loc_ref[r] - tgt
        ad = jnp.abs(d)
        sl1 = jnp.where(ad < 1.0, 0.5 * d * d, ad - 0.5)
        posf = pos.astype(jnp.float32)
        loc_sum = jnp.sum(sl1 * posf)
        n_pos = jnp.sum(posf)

        pos_ref[r] = posf
        tc_ref[r] = tc
        pm_ref[r] = jnp.concatenate(
            [jnp.full((1, 1), loc_sum), jnp.full((1, 1), n_pos),
             jnp.zeros((1, 2))], axis=1)


def _ce_body(cls_ref, tc_ref, pos_ref, pm_ref, out_ref, ceneg_ref, row_ref):
    i = pl.program_id(0)
    nsteps = pl.num_programs(0)
    P = cls_ref.shape[2]
    B = ceneg_ref.shape[0]

    for r in range(_R):
        cls = cls_ref[r]
        tc = tc_ref[r]
        posf = pos_ref[r]

        m = jnp.max(cls, axis=0, keepdims=True)
        lse = jnp.log(jnp.sum(jnp.exp(cls - m), axis=0, keepdims=True))
        cid = jax.lax.broadcasted_iota(jnp.int32, (_C, P), 0)
        logit_tc = jnp.sum(jnp.where(cid == tc, cls, 0.), axis=0,
                           keepdims=True)
        ce = m + lse - logit_tc
        row_ref[_R * i + r, 0] = jnp.sum(ce * posf)
        ceneg_ref[pl.ds(_R * i + r, 1), :] = jnp.maximum(
            jnp.where(posf > 0.5, 0., ce), 0.)

    @pl.when(i == nsteps - 1)
    def _mine():
        v = ceneg_ref[...]
        bits = jax.lax.bitcast_convert_type(v, jnp.int32)
        rid = jax.lax.broadcasted_iota(jnp.int32, (B, 1), 0)
        conf_pos = jnp.zeros((B, 1), jnp.float32)
        for r in range(B):
            conf_pos = jnp.where(rid == r, row_ref[r, 0], conf_pos)
        pm = pm_ref[...]
        loc_sum = pm[:, 0, 0:1]
        n_pos = pm[:, 0, 1:2]
        k = jnp.minimum(3 * n_pos.astype(jnp.int32), P)

        def count_ge(t):
            return jnp.sum((bits >= t).astype(jnp.int32), axis=1,
                           keepdims=True)

        def step(_, carry):
            lo, hi = carry
            q = jnp.maximum((hi - lo) // 4, 1)
            m1 = lo + q
            m2 = lo + 2 * q
            m3 = hi - q
            c1 = count_ge(m1) >= k
            c2 = count_ge(m2) >= k
            c3 = count_ge(m3) >= k
            nlo = jnp.where(c3, m3, jnp.where(c2, m2, jnp.where(c1, m1, lo)))
            nhi = jnp.where(c3, hi, jnp.where(c2, m3, jnp.where(c1, m2, m1)))
            return (nlo, nhi)

        lo0 = jnp.zeros((B, 1), jnp.int32)
        hi0 = jnp.full((B, 1), 0x7f800000, jnp.int32)
        lo, hi = jax.lax.fori_loop(0, 18, step, (lo0, hi0))
        thr = jax.lax.bitcast_convert_type(lo, jnp.float32)
        gt = bits > lo
        cnt_gt = jnp.sum(gt.astype(jnp.int32), axis=1, keepdims=True)
        sum_gt = jnp.sum(jnp.where(gt, v, 0.), axis=1, keepdims=True)
        hard = sum_gt + (k - cnt_gt).astype(jnp.float32) * thr

        n_pos_sum = jnp.sum(n_pos)
        conf_loss = (jnp.sum(hard) + jnp.sum(conf_pos)) / n_pos_sum
        loc_loss = jnp.sum(loc_sum) / n_pos_sum
        total = conf_loss + loc_loss
        out_ref[...] = jnp.concatenate(
            [jnp.full((1, 1), total), jnp.full((1, 1), loc_loss),
             jnp.full((1, 1), conf_loss), jnp.zeros((1, 1))], axis=1)


@jax.jit
def kernel(pred_loc, pred_cls, b_boxes, b_labels, priors_cxcy):
    B, P, C = pred_cls.shape
    loc_t = jnp.transpose(pred_loc, (0, 2, 1))
    cls_t = jnp.transpose(pred_cls, (0, 2, 1))
    priors_t = jnp.transpose(priors_cxcy, (1, 0))
    vals = jnp.concatenate(
        [jnp.transpose(b_boxes, (0, 2, 1)),
         b_labels.astype(jnp.float32)[:, None, :]], axis=1)

    pos, tc, pm = pl.pallas_call(
        _match_body,
        grid=(B // _R,),
        in_specs=[
            pl.BlockSpec((_R, _NOBJ, 4), lambda i: (i, 0, 0)),
            pl.BlockSpec((_R, 5, _NOBJ), lambda i: (i, 0, 0)),
            pl.BlockSpec((4, P), lambda i: (0, 0)),
            pl.BlockSpec((_R, 4, P), lambda i: (i, 0, 0)),
        ],
        out_specs=[
            pl.BlockSpec((_R, 1, P), lambda i: (i, 0, 0)),
            pl.BlockSpec((_R, 1, P), lambda i: (i, 0, 0)),
            pl.BlockSpec((_R, 1, 4), lambda i: (i, 0, 0)),
        ],
        out_shape=[
            jax.ShapeDtypeStruct((B, 1, P), jnp.float32),
            jax.ShapeDtypeStruct((B, 1, P), jnp.int32),
            jax.ShapeDtypeStruct((B, 1, 4), jnp.float32),
        ],
        scratch_shapes=[pltpu.VMEM((10, P), jnp.float32)],
    )(b_boxes, vals, priors_t, loc_t)

    out = pl.pallas_call(
        _ce_body,
        grid=(B // _R,),
        in_specs=[
            pl.BlockSpec((_R, C, P), lambda i: (i, 0, 0)),
            pl.BlockSpec((_R, 1, P), lambda i: (i, 0, 0)),
            pl.BlockSpec((_R, 1, P), lambda i: (i, 0, 0)),
            pl.BlockSpec((B, 1, 4), lambda i: (0, 0, 0)),
        ],
        out_specs=pl.BlockSpec((1, 4), lambda i: (0, 0)),
        out_shape=jax.ShapeDtypeStruct((1, 4), jnp.float32),
        scratch_shapes=[
            pltpu.VMEM((B, P), jnp.float32),
            pltpu.SMEM((B, 4), jnp.float32),
        ],
    )(cls_t, tc, pos, pm)

    return (out[0, 0], out[0, 1], out[0, 2])


# speedup vs baseline: 1.0164x; 1.0116x over previous
import jax
import jax.numpy as jnp
from jax.experimental import pallas as pl
from jax.experimental.pallas import tpu as pltpu

_C = 21
_NOBJ = 8
_R = 4


def _match_body(boxes_ref, vals_ref, priors_ref, loc_ref,
                pos_ref, tc_ref, pm_ref, pext_ref):
    i = pl.program_id(0)
    P = priors_ref.shape[1]

    @pl.when(i == 0)
    def _prep():
        pr = priors_ref[...]
        cx, cy, w, h = pr[0:1], pr[1:2], pr[2:3], pr[3:4]
        x0 = cx - w / 2.
        y0 = cy - h / 2.
        x1 = cx + w / 2.
        y1 = cy + h / 2.
        pext_ref[...] = jnp.concatenate(
            [cx, cy, w / 10., h / 10., x0, y0, x1, y1,
             (x1 - x0) * (y1 - y0), w], axis=0)

    pe = pext_ref[...]
    pcx, pcy, pw10, ph10 = pe[0:1], pe[1:2], pe[2:3], pe[3:4]
    px0, py0, px1, py1 = pe[4:5], pe[5:6], pe[6:7], pe[7:8]
    area_p = pe[8:9]
    pw, ph = pe[9:10], pe[3:4] * 10.

    for r in range(_R):
        b = boxes_ref[r]
        bx0, by0, bx1, by1 = b[:, 0:1], b[:, 1:2], b[:, 2:3], b[:, 3:4]
        area_b = (bx1 - bx0) * (by1 - by0)

        lt_x = jnp.maximum(bx0, px0)
        lt_y = jnp.maximum(by0, py0)
        rb_x = jnp.minimum(bx1, px1)
        rb_y = jnp.minimum(by1, py1)
        iw = jnp.maximum(rb_x - lt_x, 0.)
        ih = jnp.maximum(rb_y - lt_y, 0.)
        inter = iw * ih
        iou = inter / (area_b + area_p - inter)

        lane = jax.lax.broadcasted_iota(jnp.int32, (_NOBJ, P), 1)
        row_max = jnp.max(iou, axis=1, keepdims=True)
        best_p = jnp.min(jnp.where(iou == row_max, lane, P), axis=1,
                         keepdims=True)
        iou = jnp.where(lane == best_p, 1.0, iou)

        col_max = jnp.max(iou, axis=0, keepdims=True)
        pos = col_max >= 0.5
        oid = jax.lax.broadcasted_iota(jnp.int32, (_NOBJ, P), 0)
        sel = jnp.min(jnp.where(iou == col_max, oid, _NOBJ), axis=0,
                      keepdims=True)
        onehot = (oid == sel).astype(jnp.float32)

        g = jax.lax.dot_general(vals_ref[r], onehot,
                                (((1,), (0,)), ((), ())),
                                precision=jax.lax.Precision.HIGHEST,
                                preferred_element_type=jnp.float32)
        gx0, gy0, gx1, gy1 = g[0:1], g[1:2], g[2:3], g[3:4]
        tc = (g[4:5] + 0.5).astype(jnp.int32)
        tc = jnp.where(pos, tc, _C - 1)

        gcx = (gx0 + gx1) / 2.
        gcy = (gy0 + gy1) / 2.
        gw = gx1 - gx0
        gh = gy1 - gy0
        t0 = (gcx - pcx) / pw10
        t1 = (gcy - pcy) / ph10
        t2 = jnp.log(gw / pw) * 5.
        t3 = jnp.log(gh / ph) * 5.
        tgt = jnp.concatenate([t0, t1, t2, t3], axis=0)

        d = loc_ref[r] - tgt
        ad = jnp.abs(d)
        sl1 = jnp.where(ad < 1.0, 0.5 * d * d, ad - 0.5)
        posf = pos.astype(jnp.float32)
        loc_sum = jnp.sum(sl1 * posf)
        n_pos = jnp.sum(posf)

        pos_ref[r] = posf
        tc_ref[r] = tc
        pm_ref[r] = jnp.concatenate(
            [jnp.full((1, 1), loc_sum), jnp.full((1, 1), n_pos),
             jnp.zeros((1, 2))], axis=1)


def _ce_body(cls_ref, tc_ref, pos_ref, pm_ref, out_ref, ceneg_ref, row_ref):
    i = pl.program_id(0)
    nsteps = pl.num_programs(0)
    P = cls_ref.shape[2]
    B = ceneg_ref.shape[0]

    for r in range(_R):
        cls = cls_ref[r]
        tc = tc_ref[r]
        posf = pos_ref[r]

        m = jnp.max(cls, axis=0, keepdims=True)
        lse = jnp.log(jnp.sum(jnp.exp(cls - m), axis=0, keepdims=True))
        cid = jax.lax.broadcasted_iota(jnp.int32, (_C, P), 0)
        logit_tc = jnp.sum(jnp.where(cid == tc, cls, 0.), axis=0,
                           keepdims=True)
        ce = m + lse - logit_tc
        row_ref[_R * i + r, 0] = jnp.sum(ce * posf)
        ceneg_ref[pl.ds(_R * i + r, 1), :] = jnp.maximum(
            jnp.where(posf > 0.5, 0., ce), 0.)

    @pl.when(i == nsteps - 1)
    def _mine():
        v = ceneg_ref[...]
        bits = jax.lax.bitcast_convert_type(v, jnp.int32)
        rid = jax.lax.broadcasted_iota(jnp.int32, (B, 1), 0)
        conf_pos = jnp.zeros((B, 1), jnp.float32)
        for r in range(B):
            conf_pos = jnp.where(rid == r, row_ref[r, 0], conf_pos)
        pm = pm_ref[...]
        loc_sum = pm[:, 0, 0:1]
        n_pos = pm[:, 0, 1:2]
        k = jnp.minimum(3 * n_pos.astype(jnp.int32), P)

        def count_ge(t):
            return jnp.sum((bits >= t).astype(jnp.int32), axis=1,
                           keepdims=True)

        def step(_, carry):
            lo, hi = carry
            q = jnp.maximum((hi - lo) // 4, 1)
            m1 = lo + q
            m2 = lo + 2 * q
            m3 = hi - q
            c1 = count_ge(m1) >= k
            c2 = count_ge(m2) >= k
            c3 = count_ge(m3) >= k
            nlo = jnp.where(c3, m3, jnp.where(c2, m2, jnp.where(c1, m1, lo)))
            nhi = jnp.where(c3, hi, jnp.where(c2, m3, jnp.where(c1, m2, m1)))
            return (nlo, nhi)

        lo = jnp.zeros((B, 1), jnp.int32)
        hi = jnp.full((B, 1), 0x7f800000, jnp.int32)
        for _ in range(18):
            lo, hi = step(0, (lo, hi))
        thr = jax.lax.bitcast_convert_type(lo, jnp.float32)
        gt = bits > lo
        cnt_gt = jnp.sum(gt.astype(jnp.int32), axis=1, keepdims=True)
        sum_gt = jnp.sum(jnp.where(gt, v, 0.), axis=1, keepdims=True)
        hard = sum_gt + (k - cnt_gt).astype(jnp.float32) * thr

        n_pos_sum = jnp.sum(n_pos)
        conf_loss = (jnp.sum(hard) + jnp.sum(conf_pos)) / n_pos_sum
        loc_loss = jnp.sum(loc_sum) / n_pos_sum
        total = conf_loss + loc_loss
        out_ref[...] = jnp.concatenate(
            [jnp.full((1, 1), total), jnp.full((1, 1), loc_loss),
             jnp.full((1, 1), conf_loss), jnp.zeros((1, 1))], axis=1)


@jax.jit
def kernel(pred_loc, pred_cls, b_boxes, b_labels, priors_cxcy):
    B, P, C = pred_cls.shape
    loc_t = jnp.transpose(pred_loc, (0, 2, 1))
    cls_t = jnp.transpose(pred_cls, (0, 2, 1))
    priors_t = jnp.transpose(priors_cxcy, (1, 0))
    vals = jnp.concatenate(
        [jnp.transpose(b_boxes, (0, 2, 1)),
         b_labels.astype(jnp.float32)[:, None, :]], axis=1)

    pos, tc, pm = pl.pallas_call(
        _match_body,
        grid=(B // _R,),
        in_specs=[
            pl.BlockSpec((_R, _NOBJ, 4), lambda i: (i, 0, 0)),
            pl.BlockSpec((_R, 5, _NOBJ), lambda i: (i, 0, 0)),
            pl.BlockSpec((4, P), lambda i: (0, 0)),
            pl.BlockSpec((_R, 4, P), lambda i: (i, 0, 0)),
        ],
        out_specs=[
            pl.BlockSpec((_R, 1, P), lambda i: (i, 0, 0)),
            pl.BlockSpec((_R, 1, P), lambda i: (i, 0, 0)),
            pl.BlockSpec((_R, 1, 4), lambda i: (i, 0, 0)),
        ],
        out_shape=[
            jax.ShapeDtypeStruct((B, 1, P), jnp.float32),
            jax.ShapeDtypeStruct((B, 1, P), jnp.int32),
            jax.ShapeDtypeStruct((B, 1, 4), jnp.float32),
        ],
        scratch_shapes=[pltpu.VMEM((10, P), jnp.float32)],
    )(b_boxes, vals, priors_t, loc_t)

    out = pl.pallas_call(
        _ce_body,
        grid=(B // _R,),
        in_specs=[
            pl.BlockSpec((_R, C, P), lambda i: (i, 0, 0)),
            pl.BlockSpec((_R, 1, P), lambda i: (i, 0, 0)),
            pl.BlockSpec((_R, 1, P), lambda i: (i, 0, 0)),
            pl.BlockSpec((B, 1, 4), lambda i: (0, 0, 0)),
        ],
        out_specs=pl.BlockSpec((1, 4), lambda i: (0, 0)),
        out_shape=jax.ShapeDtypeStruct((1, 4), jnp.float32),
        scratch_shapes=[
            pltpu.VMEM((B, P), jnp.float32),
            pltpu.SMEM((B, 4), jnp.float32),
        ],
    )(cls_t, tc, pos, pm)

    return (out[0, 0], out[0, 1], out[0, 2])
